# Initial kernel scaffold; baseline (speedup 1.0000x reference)
import functools

import jax
import jax.numpy as jnp
from jax import lax
from jax.experimental import pallas as pl
from jax.experimental.pallas import tpu as pltpu
from jax.experimental.pallas import tpu_sc as plsc

NC = 2
NS = 16
L = 16
CH = 128


def _tc_matmul_body(xs_ref, xd_ref, wnT_ref, wsT_ref, b_ref, h_ref, o_ref):
    h_ref[...] = jax.lax.dot_general(
        xs_ref[...], wnT_ref[...], (((1,), (0,)), ((), ())),
        preferred_element_type=jnp.float32)
    o_ref[...] = jax.lax.dot_general(
        xd_ref[...], wsT_ref[...], (((1,), (0,)), ((), ())),
        preferred_element_type=jnp.float32) + b_ref[...]


def _combine_body(p_ref, o_ref):
    o_ref[...] = p_ref[0] + p_ref[1]


def _make_sc_kernel(N, D, EPT):
    NCH = EPT // CH
    mesh = plsc.VectorSubcoreMesh(
        core_axis_name="c", subcore_axis_name="s",
        num_cores=NC, num_subcores=NS)

    @functools.partial(
        pl.kernel,
        out_type=jax.ShapeDtypeStruct((NC, N, D), jnp.float32),
        mesh=mesh,
        scratch_types=[
            pltpu.VMEM((EPT,), jnp.int32),
            pltpu.VMEM((EPT,), jnp.float32),
            pltpu.VMEM((NCH, CH), jnp.int32),
            pltpu.VMEM((CH, D), jnp.float32),
            pltpu.VMEM_SHARED((N, D), jnp.float32),
            pltpu.SemaphoreType.DMA,
        ],
    )
    def sc_kernel(h_hbm, init2_hbm, src_hbm, w_hbm, dst2_hbm, out_hbm,
                  srcv, wv, dstv, rows, acc, sem):
        c = lax.axis_index("c")
        s = lax.axis_index("s")
        wid = c * NS + s

        ebase = wid * EPT
        pltpu.sync_copy(src_hbm.at[pl.ds(ebase, EPT)], srcv)
        pltpu.sync_copy(w_hbm.at[pl.ds(ebase, EPT)], wv)
        pltpu.sync_copy(dst2_hbm.at[pl.ds(wid * NCH, NCH)], dstv)
        rpt = N // NS
        rbase = s * rpt
        pltpu.sync_copy(init2_hbm.at[c].at[pl.ds(rbase, rpt)],
                        acc.at[pl.ds(rbase, rpt)])
        plsc.subcore_barrier()

        def chunk_body(k, carry):
            pltpu.async_copy(
                h_hbm.at[srcv.at[pl.ds(k * CH, CH)]], rows, sem).wait()

            def edge_body(i, carry2):
                w = wv[k * CH + i]
                wb = jnp.full((L,), w, jnp.float32)
                for j in range(D // L):
                    sl = pl.ds(j * L, L)
                    rows[i, sl] = rows[i, sl] * wb
                return carry2

            lax.fori_loop(0, CH, edge_body, 0)
            pltpu.sync_copy(rows, acc.at[dstv.at[k]], add=True)
            return carry

        lax.fori_loop(0, NCH, chunk_body, 0)
        plsc.subcore_barrier()

        pltpu.sync_copy(acc.at[pl.ds(rbase, rpt)],
                        out_hbm.at[c].at[pl.ds(rbase, rpt)])

    return sc_kernel


def kernel(x_src, x_dst, edge_index_sd, edge_weight, W_nei, W_self, b_self):
    N_SRC, D = x_src.shape
    N_DST = x_dst.shape[0]
    E = edge_weight.shape[0]
    NW = NC * NS

    EPT = ((E + NW * CH - 1) // (NW * CH)) * CH
    E_pad = EPT * NW
    src = edge_index_sd[0].astype(jnp.int32)
    dst = edge_index_sd[1].astype(jnp.int32)
    pad = E_pad - E
    src_p = jnp.concatenate([src, jnp.zeros((pad,), jnp.int32)])
    dst_p = jnp.concatenate([dst, jnp.zeros((pad,), jnp.int32)])
    w_p = jnp.concatenate([edge_weight, jnp.zeros((pad,), jnp.float32)])
    dst2 = dst_p.reshape(E_pad // CH, CH)

    BN = 1000
    h, out0 = pl.pallas_call(
        _tc_matmul_body,
        grid=(N_SRC // BN,),
        in_specs=[
            pl.BlockSpec((BN, D), lambda i: (i, 0)),
            pl.BlockSpec((BN, D), lambda i: (i, 0)),
            pl.BlockSpec((D, D), lambda i: (0, 0)),
            pl.BlockSpec((D, D), lambda i: (0, 0)),
            pl.BlockSpec((1, D), lambda i: (0, 0)),
        ],
        out_specs=[
            pl.BlockSpec((BN, D), lambda i: (i, 0)),
            pl.BlockSpec((BN, D), lambda i: (i, 0)),
        ],
        out_shape=[
            jax.ShapeDtypeStruct((N_SRC, D), jnp.float32),
            jax.ShapeDtypeStruct((N_DST, D), jnp.float32),
        ],
    )(x_src, x_dst, W_nei.T, W_self.T, b_self.reshape(1, D))

    init2 = jnp.stack([out0, jnp.zeros_like(out0)])

    sc_kernel = _make_sc_kernel(N_DST, D, EPT)
    partials = sc_kernel(h, init2, src_p, w_p, dst2)

    out = pl.pallas_call(
        _combine_body,
        grid=(N_DST // BN,),
        in_specs=[pl.BlockSpec((NC, BN, D), lambda i: (0, i, 0))],
        out_specs=pl.BlockSpec((BN, D), lambda i: (i, 0)),
        out_shape=jax.ShapeDtypeStruct((N_DST, D), jnp.float32),
    )(partials)
    return out


# speedup vs baseline: 3.3148x; 3.3148x over previous
import functools

import jax
import jax.numpy as jnp
from jax import lax
from jax.experimental import pallas as pl
from jax.experimental.pallas import tpu as pltpu
from jax.experimental.pallas import tpu_sc as plsc

NC = 2
NS = 16
L = 16
CH = 128


def _tc_matmul_body(xs_ref, xd_ref, wnT_ref, wsT_ref, b_ref, h_ref, o_ref):
    h_ref[...] = jax.lax.dot_general(
        xs_ref[...], wnT_ref[...], (((1,), (0,)), ((), ())),
        preferred_element_type=jnp.float32)
    o_ref[...] = jax.lax.dot_general(
        xd_ref[...], wsT_ref[...], (((1,), (0,)), ((), ())),
        preferred_element_type=jnp.float32) + b_ref[...]


def _combine_body(p_ref, o_ref):
    o_ref[...] = p_ref[0] + p_ref[1]


def _make_sc_kernel(N, D, EPT):
    NCH = EPT // CH
    mesh = plsc.VectorSubcoreMesh(
        core_axis_name="c", subcore_axis_name="s",
        num_cores=NC, num_subcores=NS)

    @functools.partial(
        pl.kernel,
        out_type=jax.ShapeDtypeStruct((NC, N, D), jnp.float32),
        mesh=mesh,
        scratch_types=[
            pltpu.VMEM((EPT,), jnp.int32),
            pltpu.VMEM((EPT,), jnp.float32),
            pltpu.VMEM((NCH, CH), jnp.int32),
            pltpu.VMEM((CH, D), jnp.float32),
            pltpu.VMEM_SHARED((N, D), jnp.float32),
            pltpu.SemaphoreType.DMA,
        ],
    )
    def sc_kernel(h_hbm, init2_hbm, src_hbm, w_hbm, dst2_hbm, out_hbm,
                  srcv, wv, dstv, rows, acc, sem):
        c = lax.axis_index("c")
        s = lax.axis_index("s")
        wid = c * NS + s

        RPT = (-(-N // NS) + 7) // 8 * 8
        RPT_LAST = N - (NS - 1) * RPT

        ebase = pl.multiple_of(wid * EPT, 8)
        pltpu.sync_copy(src_hbm.at[pl.ds(ebase, EPT)], srcv)
        pltpu.sync_copy(w_hbm.at[pl.ds(ebase, EPT)], wv)
        pltpu.sync_copy(dst2_hbm.at[pl.ds(pl.multiple_of(wid * NCH, 8), NCH)],
                        dstv)
        rbase = pl.multiple_of(s * RPT, 8)

        @pl.when(s < NS - 1)
        def _():
            pltpu.sync_copy(init2_hbm.at[c].at[pl.ds(rbase, RPT)],
                            acc.at[pl.ds(rbase, RPT)])

        @pl.when(s == NS - 1)
        def _():
            pltpu.sync_copy(
                init2_hbm.at[c].at[pl.ds((NS - 1) * RPT, RPT_LAST)],
                acc.at[pl.ds((NS - 1) * RPT, RPT_LAST)])

        plsc.subcore_barrier()

        def chunk_body(k, carry):
            pltpu.async_copy(
                h_hbm.at[srcv.at[pl.ds(k * CH, CH)]], rows, sem).wait()

            def group_body(g, carry2):
                w16 = wv[pl.ds(k * CH + g * L, L)]
                for e in range(L):
                    wb = jnp.full((L,), w16[e], jnp.float32)
                    row = g * L + e
                    for j in range(D // L):
                        sl = pl.ds(j * L, L)
                        rows[row, sl] = rows[row, sl] * wb
                return carry2

            lax.fori_loop(0, CH // L, group_body, 0)
            pltpu.sync_copy(rows, acc.at[dstv.at[k]], add=True)
            return carry

        lax.fori_loop(0, NCH, chunk_body, 0)
        plsc.subcore_barrier()

        @pl.when(s < NS - 1)
        def _():
            pltpu.sync_copy(acc.at[pl.ds(rbase, RPT)],
                            out_hbm.at[c].at[pl.ds(rbase, RPT)])

        @pl.when(s == NS - 1)
        def _():
            pltpu.sync_copy(
                acc.at[pl.ds((NS - 1) * RPT, RPT_LAST)],
                out_hbm.at[c].at[pl.ds((NS - 1) * RPT, RPT_LAST)])

    return sc_kernel


def kernel(x_src, x_dst, edge_index_sd, edge_weight, W_nei, W_self, b_self):
    N_SRC, D = x_src.shape
    N_DST = x_dst.shape[0]
    E = edge_weight.shape[0]
    NW = NC * NS

    EPT = ((E + NW * CH * 8 - 1) // (NW * CH * 8)) * CH * 8
    E_pad = EPT * NW
    src = edge_index_sd[0].astype(jnp.int32)
    dst = edge_index_sd[1].astype(jnp.int32)
    pad = E_pad - E
    src_p = jnp.concatenate([src, jnp.zeros((pad,), jnp.int32)])
    dst_p = jnp.concatenate([dst, jnp.zeros((pad,), jnp.int32)])
    w_p = jnp.concatenate([edge_weight, jnp.zeros((pad,), jnp.float32)])
    dst2 = dst_p.reshape(E_pad // CH, CH)

    BN = 1000
    h, out0 = pl.pallas_call(
        _tc_matmul_body,
        grid=(N_SRC // BN,),
        in_specs=[
            pl.BlockSpec((BN, D), lambda i: (i, 0)),
            pl.BlockSpec((BN, D), lambda i: (i, 0)),
            pl.BlockSpec((D, D), lambda i: (0, 0)),
            pl.BlockSpec((D, D), lambda i: (0, 0)),
            pl.BlockSpec((1, D), lambda i: (0, 0)),
        ],
        out_specs=[
            pl.BlockSpec((BN, D), lambda i: (i, 0)),
            pl.BlockSpec((BN, D), lambda i: (i, 0)),
        ],
        out_shape=[
            jax.ShapeDtypeStruct((N_SRC, D), jnp.float32),
            jax.ShapeDtypeStruct((N_DST, D), jnp.float32),
        ],
    )(x_src, x_dst, W_nei.T, W_self.T, b_self.reshape(1, D))

    init2 = jnp.stack([out0, jnp.zeros_like(out0)])

    sc_kernel = _make_sc_kernel(N_DST, D, EPT)
    partials = sc_kernel(h, init2, src_p, w_p, dst2)

    out = pl.pallas_call(
        _combine_body,
        grid=(N_DST // BN,),
        in_specs=[pl.BlockSpec((NC, BN, D), lambda i: (0, i, 0))],
        out_specs=pl.BlockSpec((BN, D), lambda i: (i, 0)),
        out_shape=jax.ShapeDtypeStruct((N_DST, D), jnp.float32),
    )(partials)
    return out


# speedup vs baseline: 8.4764x; 2.5571x over previous
import functools

import jax
import jax.numpy as jnp
from jax import lax
from jax.experimental import pallas as pl
from jax.experimental.pallas import tpu as pltpu
from jax.experimental.pallas import tpu_sc as plsc

NC = 2
NS = 16
L = 16
CH = 32


def _tc_matmul_body(xs_ref, xd_ref, wnT_ref, wsT_ref, b_ref, h_ref, o_ref):
    h_ref[...] = jax.lax.dot_general(
        xs_ref[...], wnT_ref[...], (((1,), (0,)), ((), ())),
        preferred_element_type=jnp.float32)
    o_ref[...] = jax.lax.dot_general(
        xd_ref[...], wsT_ref[...], (((1,), (0,)), ((), ())),
        preferred_element_type=jnp.float32) + b_ref[...]


def _combine_body(p_ref, o_ref):
    o_ref[...] = p_ref[0] + p_ref[1]


def _make_sc_kernel(N, D, EPT):
    NCH = EPT // CH
    mesh = plsc.VectorSubcoreMesh(
        core_axis_name="c", subcore_axis_name="s",
        num_cores=NC, num_subcores=NS)

    @functools.partial(
        pl.kernel,
        out_type=jax.ShapeDtypeStruct((NC, N, D), jnp.float32),
        mesh=mesh,
        scratch_types=[
            pltpu.VMEM((EPT,), jnp.int32),
            pltpu.VMEM((EPT,), jnp.float32),
            pltpu.VMEM((EPT // D, D), jnp.int32),
            [pltpu.VMEM((CH, D), jnp.float32) for _ in range(4)],
            [pltpu.VMEM((CH,), jnp.int32) for _ in range(4)],
            [pltpu.SemaphoreType.DMA for _ in range(4)],
            [pltpu.SemaphoreType.DMA for _ in range(4)],
            pltpu.VMEM_SHARED((N, D), jnp.float32),
        ],
    )
    def sc_kernel(h_hbm, init2_hbm, src_hbm, w_hbm, dst2_hbm, out_hbm,
                  srcv, wv, dstv, rows, dbuf, gsem, csem, acc):
        c = lax.axis_index("c")
        s = lax.axis_index("s")
        wid = c * NS + s

        RPT = (-(-N // NS) + 7) // 8 * 8
        RPT_LAST = N - (NS - 1) * RPT

        ebase = pl.multiple_of(wid * EPT, 8)
        pltpu.sync_copy(src_hbm.at[pl.ds(ebase, EPT)], srcv)
        pltpu.sync_copy(w_hbm.at[pl.ds(ebase, EPT)], wv)
        DR = EPT // D
        pltpu.sync_copy(dst2_hbm.at[pl.ds(pl.multiple_of(wid * DR, 8), DR)],
                        dstv)
        rbase = pl.multiple_of(s * RPT, 8)

        @pl.when(s < NS - 1)
        def _():
            pltpu.sync_copy(init2_hbm.at[c].at[pl.ds(rbase, RPT)],
                            acc.at[pl.ds(rbase, RPT)])

        @pl.when(s == NS - 1)
        def _():
            pltpu.sync_copy(
                init2_hbm.at[c].at[pl.ds((NS - 1) * RPT, RPT_LAST)],
                acc.at[pl.ds((NS - 1) * RPT, RPT_LAST)])

        plsc.subcore_barrier()

        def issue_gather(k, b):
            pltpu.async_copy(
                h_hbm.at[srcv.at[pl.ds(k * CH, CH)]], rows[b], gsem[b])

        def wait_gather(k, b):
            pltpu.make_async_copy(
                h_hbm.at[srcv.at[pl.ds(k * CH, CH)]], rows[b],
                gsem[b]).wait()

        CPR = D // CH

        def fill_dbuf(k, b):
            r = k // CPR
            col = (k % CPR) * CH
            for q in range(CH // L):
                dbuf[b][pl.ds(q * L, L)] = dstv[r, pl.ds(col + q * L, L)]

        def issue_scatter(k, b):
            pltpu.async_copy(rows[b], acc.at[dbuf[b]], csem[b], add=True)

        def wait_scatter(b):
            pltpu.make_async_copy(rows[b], acc.at[dbuf[b]], csem[b]).wait()

        def scale(k, b):
            def group_body(g, carry2):
                w16 = wv[pl.ds(k * CH + g * L, L)]
                for e in range(L):
                    wb = jnp.full((L,), w16[e], jnp.float32)
                    row = g * L + e
                    for j in range(D // L):
                        sl = pl.ds(j * L, L)
                        rows[b][row, sl] = rows[b][row, sl] * wb
                return carry2

            lax.fori_loop(0, CH // L, group_body, 0)

        issue_gather(0, 0)
        issue_gather(1, 1)

        def quad_body(q, carry):
            k0 = q * 4
            for b in range(4):
                k = k0 + b
                wait_gather(k, b)
                scale(k, b)
                fill_dbuf(k, b)
                issue_scatter(k, b)
                b2 = (b + 2) % 4

                @pl.when(k >= 2)
                def _():
                    wait_scatter(b2)

                @pl.when(k < NCH - 2)
                def _():
                    issue_gather(k + 2, b2)
            return carry

        lax.fori_loop(0, NCH // 4, quad_body, 0)
        wait_scatter((NCH - 2) % 4)
        wait_scatter((NCH - 1) % 4)
        plsc.subcore_barrier()

        @pl.when(s < NS - 1)
        def _():
            pltpu.sync_copy(acc.at[pl.ds(rbase, RPT)],
                            out_hbm.at[c].at[pl.ds(rbase, RPT)])

        @pl.when(s == NS - 1)
        def _():
            pltpu.sync_copy(
                acc.at[pl.ds((NS - 1) * RPT, RPT_LAST)],
                out_hbm.at[c].at[pl.ds((NS - 1) * RPT, RPT_LAST)])

    return sc_kernel


def kernel(x_src, x_dst, edge_index_sd, edge_weight, W_nei, W_self, b_self):
    N_SRC, D = x_src.shape
    N_DST = x_dst.shape[0]
    E = edge_weight.shape[0]
    NW = NC * NS

    EPT = ((E + NW * CH * 8 - 1) // (NW * CH * 8)) * CH * 8
    E_pad = EPT * NW
    src = edge_index_sd[0].astype(jnp.int32)
    dst = edge_index_sd[1].astype(jnp.int32)
    pad = E_pad - E
    pad_idx = jnp.arange(pad, dtype=jnp.int32)
    src_p = jnp.concatenate([src, pad_idx % N_SRC])
    dst_p = jnp.concatenate([dst, pad_idx % N_DST])
    w_p = jnp.concatenate([edge_weight, jnp.zeros((pad,), jnp.float32)])
    dst2 = dst_p.reshape(E_pad // D, D)

    BN = 1000
    h, out0 = pl.pallas_call(
        _tc_matmul_body,
        grid=(N_SRC // BN,),
        in_specs=[
            pl.BlockSpec((BN, D), lambda i: (i, 0)),
            pl.BlockSpec((BN, D), lambda i: (i, 0)),
            pl.BlockSpec((D, D), lambda i: (0, 0)),
            pl.BlockSpec((D, D), lambda i: (0, 0)),
            pl.BlockSpec((1, D), lambda i: (0, 0)),
        ],
        out_specs=[
            pl.BlockSpec((BN, D), lambda i: (i, 0)),
            pl.BlockSpec((BN, D), lambda i: (i, 0)),
        ],
        out_shape=[
            jax.ShapeDtypeStruct((N_SRC, D), jnp.float32),
            jax.ShapeDtypeStruct((N_DST, D), jnp.float32),
        ],
    )(x_src, x_dst, W_nei.T, W_self.T, b_self.reshape(1, D))

    init2 = jnp.stack([out0, jnp.zeros_like(out0)])

    sc_kernel = _make_sc_kernel(N_DST, D, EPT)
    partials = sc_kernel(h, init2, src_p, w_p, dst2)

    out = pl.pallas_call(
        _combine_body,
        grid=(N_DST // BN,),
        in_specs=[pl.BlockSpec((NC, BN, D), lambda i: (0, i, 0))],
        out_specs=pl.BlockSpec((BN, D), lambda i: (i, 0)),
        out_shape=jax.ShapeDtypeStruct((N_DST, D), jnp.float32),
    )(partials)
    return out


# speedup vs baseline: 10.3904x; 1.2258x over previous
import functools

import jax
import jax.numpy as jnp
from jax import lax
from jax.experimental import pallas as pl
from jax.experimental.pallas import tpu as pltpu
from jax.experimental.pallas import tpu_sc as plsc

NC = 2
NS = 16
L = 16
CH = 64
NW_ = 2


def _tc_matmul_body(xs_ref, xd_ref, wnT_ref, wsT_ref, b_ref, h_ref, o_ref):
    h_ref[...] = jax.lax.dot_general(
        xs_ref[...], wnT_ref[...], (((1,), (0,)), ((), ())),
        preferred_element_type=jnp.float32)
    o_ref[...] = jax.lax.dot_general(
        xd_ref[...], wsT_ref[...], (((1,), (0,)), ((), ())),
        preferred_element_type=jnp.float32) + b_ref[...]


def _combine_body(p_ref, o_ref):
    o_ref[...] = p_ref[0] + p_ref[1]


def _make_sc_kernel(N, D, EPT):
    EPW = EPT // NW_
    WCH = EPW // CH
    mesh = plsc.VectorSubcoreMesh(
        core_axis_name="c", subcore_axis_name="s",
        num_cores=NC, num_subcores=NS)

    @functools.partial(
        pl.kernel,
        out_type=jax.ShapeDtypeStruct((NC, N, D), jnp.float32),
        mesh=mesh,
        scratch_types=[
            pltpu.VMEM((EPW,), jnp.int32),
            pltpu.VMEM((EPW,), jnp.float32),
            pltpu.VMEM((EPW // D, D), jnp.int32),
            [pltpu.VMEM((CH, D), jnp.float32) for _ in range(4)],
            [pltpu.VMEM((CH,), jnp.int32) for _ in range(4)],
            [pltpu.SemaphoreType.DMA for _ in range(4)],
            [pltpu.SemaphoreType.DMA for _ in range(4)],
            pltpu.VMEM_SHARED((N, D), jnp.float32),
        ],
    )
    def sc_kernel(h_hbm, init2_hbm, src_hbm, w_hbm, dst2_hbm, out_hbm,
                  srcv, wv, dstv, rows, dbuf, gsem, csem, acc):
        c = lax.axis_index("c")
        s = lax.axis_index("s")
        wid = c * NS + s

        RPT = (-(-N // NS) + 7) // 8 * 8
        RPT_LAST = N - (NS - 1) * RPT

        DR = EPW // D

        def stage(win):
            base = pl.multiple_of(wid * EPT + win * EPW, 8)
            dbase = pl.multiple_of(wid * (EPT // D) + win * DR, 8)
            c1 = pltpu.async_copy(src_hbm.at[pl.ds(base, EPW)], srcv,
                                  gsem[0])
            c2 = pltpu.async_copy(w_hbm.at[pl.ds(base, EPW)], wv, gsem[1])
            c3 = pltpu.async_copy(dst2_hbm.at[pl.ds(dbase, DR)], dstv,
                                  gsem[2])
            c1.wait()
            c2.wait()
            c3.wait()

        stage(0)
        pltpu.async_copy(h_hbm.at[srcv.at[pl.ds(0, CH)]], rows[0], gsem[0])
        pltpu.async_copy(h_hbm.at[srcv.at[pl.ds(CH, CH)]], rows[1], gsem[1])
        rbase = pl.multiple_of(s * RPT, 8)

        @pl.when(s < NS - 1)
        def _():
            pltpu.sync_copy(init2_hbm.at[c].at[pl.ds(rbase, RPT)],
                            acc.at[pl.ds(rbase, RPT)])

        @pl.when(s == NS - 1)
        def _():
            pltpu.sync_copy(
                init2_hbm.at[c].at[pl.ds((NS - 1) * RPT, RPT_LAST)],
                acc.at[pl.ds((NS - 1) * RPT, RPT_LAST)])

        plsc.subcore_barrier()

        def issue_gather(k, b):
            pltpu.async_copy(
                h_hbm.at[srcv.at[pl.ds(k * CH, CH)]], rows[b], gsem[b])

        def wait_gather(k, b):
            pltpu.make_async_copy(
                h_hbm.at[srcv.at[pl.ds(k * CH, CH)]], rows[b],
                gsem[b]).wait()

        CPR = D // CH

        def fill_dbuf(k, b):
            r = k // CPR
            col = (k % CPR) * CH
            for q in range(CH // L):
                dbuf[b][pl.ds(q * L, L)] = dstv[r, pl.ds(col + q * L, L)]

        def issue_scatter(k, b):
            pltpu.async_copy(rows[b], acc.at[dbuf[b]], csem[b], add=True)

        def wait_scatter(b):
            pltpu.make_async_copy(rows[b], acc.at[dbuf[b]], csem[b]).wait()

        def scale(k, b):
            def group_body(g, carry2):
                w16 = wv[pl.ds(k * CH + g * L, L)]
                for e in range(L):
                    wb = jnp.full((L,), w16[e], jnp.float32)
                    row = g * L + e
                    for j in range(D // L):
                        sl = pl.ds(j * L, L)
                        rows[b][row, sl] = rows[b][row, sl] * wb
                return carry2

            lax.fori_loop(0, CH // L, group_body, 0)

        def quad_body(q, carry):
            k0 = q * 4
            for b in range(4):
                k = k0 + b
                wait_gather(k, b)
                scale(k, b)
                fill_dbuf(k, b)
                issue_scatter(k, b)
                b2 = (b + 2) % 4

                @pl.when(k >= 2)
                def _():
                    wait_scatter(b2)

                @pl.when(k < WCH - 2)
                def _():
                    issue_gather(k + 2, b2)
            return carry

        for win in range(NW_):
            if win > 0:
                stage(win)
                issue_gather(0, 0)
                issue_gather(1, 1)
            lax.fori_loop(0, WCH // 4, quad_body, 0)
            wait_scatter((WCH - 2) % 4)
            wait_scatter((WCH - 1) % 4)
        plsc.subcore_barrier()

        @pl.when(s < NS - 1)
        def _():
            pltpu.sync_copy(acc.at[pl.ds(rbase, RPT)],
                            out_hbm.at[c].at[pl.ds(rbase, RPT)])

        @pl.when(s == NS - 1)
        def _():
            pltpu.sync_copy(
                acc.at[pl.ds((NS - 1) * RPT, RPT_LAST)],
                out_hbm.at[c].at[pl.ds((NS - 1) * RPT, RPT_LAST)])

    return sc_kernel


def kernel(x_src, x_dst, edge_index_sd, edge_weight, W_nei, W_self, b_self):
    N_SRC, D = x_src.shape
    N_DST = x_dst.shape[0]
    E = edge_weight.shape[0]
    NW = NC * NS

    EPT = ((E + NW * CH * 8 - 1) // (NW * CH * 8)) * CH * 8
    E_pad = EPT * NW
    src = edge_index_sd[0].astype(jnp.int32)
    dst = edge_index_sd[1].astype(jnp.int32)
    pad = E_pad - E
    pad_idx = jnp.arange(pad, dtype=jnp.int32)
    src_p = jnp.concatenate([src, pad_idx % N_SRC])
    dst_p = jnp.concatenate([dst, pad_idx % N_DST])
    w_p = jnp.concatenate([edge_weight, jnp.zeros((pad,), jnp.float32)])
    dst2 = dst_p.reshape(E_pad // D, D)

    BN = 1000
    h, out0 = pl.pallas_call(
        _tc_matmul_body,
        grid=(N_SRC // BN,),
        in_specs=[
            pl.BlockSpec((BN, D), lambda i: (i, 0)),
            pl.BlockSpec((BN, D), lambda i: (i, 0)),
            pl.BlockSpec((D, D), lambda i: (0, 0)),
            pl.BlockSpec((D, D), lambda i: (0, 0)),
            pl.BlockSpec((1, D), lambda i: (0, 0)),
        ],
        out_specs=[
            pl.BlockSpec((BN, D), lambda i: (i, 0)),
            pl.BlockSpec((BN, D), lambda i: (i, 0)),
        ],
        out_shape=[
            jax.ShapeDtypeStruct((N_SRC, D), jnp.float32),
            jax.ShapeDtypeStruct((N_DST, D), jnp.float32),
        ],
    )(x_src, x_dst, W_nei.T, W_self.T, b_self.reshape(1, D))

    init2 = jnp.stack([out0, jnp.zeros_like(out0)])

    sc_kernel = _make_sc_kernel(N_DST, D, EPT)
    partials = sc_kernel(h, init2, src_p, w_p, dst2)

    out = pl.pallas_call(
        _combine_body,
        grid=(N_DST // BN,),
        in_specs=[pl.BlockSpec((NC, BN, D), lambda i: (0, i, 0))],
        out_specs=pl.BlockSpec((BN, D), lambda i: (i, 0)),
        out_shape=jax.ShapeDtypeStruct((N_DST, D), jnp.float32),
    )(partials)
    return out


# speedup vs baseline: 11.5572x; 1.1123x over previous
import functools

import jax
import jax.numpy as jnp
from jax import lax
from jax.experimental import pallas as pl
from jax.experimental.pallas import tpu as pltpu
from jax.experimental.pallas import tpu_sc as plsc

NC = 2
NS = 16
L = 16
CH = 64
NW_ = 2


def _tc_matmul_body(xs_ref, xd_ref, wn_ref, ws_ref, b_ref, h_ref, o_ref):
    h_ref[...] = jax.lax.dot_general(
        xs_ref[...], wn_ref[...], (((1,), (1,)), ((), ())),
        preferred_element_type=jnp.float32)
    o_ref[...] = jax.lax.dot_general(
        xd_ref[...], ws_ref[...], (((1,), (1,)), ((), ())),
        preferred_element_type=jnp.float32) + b_ref[...]


def _combine_body(p_ref, o_ref):
    o_ref[...] = p_ref[0] + p_ref[1]


def _make_sc_kernel(N, D, EPT):
    EPW = EPT // NW_
    WCH = EPW // CH
    mesh = plsc.VectorSubcoreMesh(
        core_axis_name="c", subcore_axis_name="s",
        num_cores=NC, num_subcores=NS)

    @functools.partial(
        pl.kernel,
        out_type=jax.ShapeDtypeStruct((NC, N, D), jnp.float32),
        mesh=mesh,
        scratch_types=[
            pltpu.VMEM((EPW,), jnp.int32),
            pltpu.VMEM((EPW,), jnp.float32),
            pltpu.VMEM((EPW,), jnp.int32),
            [pltpu.VMEM((CH, D), jnp.float32) for _ in range(4)],
            [pltpu.VMEM((CH,), jnp.int32) for _ in range(4)],
            [pltpu.SemaphoreType.DMA for _ in range(4)],
            [pltpu.SemaphoreType.DMA for _ in range(4)],
            pltpu.VMEM_SHARED((N, D), jnp.float32),
        ],
    )
    def sc_kernel(h_hbm, out0_hbm, src_hbm, w_hbm, dst_hbm, out_hbm,
                  srcv, wv, dstv, rows, dbuf, gsem, csem, acc):
        c = lax.axis_index("c")
        s = lax.axis_index("s")
        wid = c * NS + s

        RPT = (-(-N // NS) + 7) // 8 * 8
        RPT_LAST = N - (NS - 1) * RPT

        def stage(win):
            base = pl.multiple_of(wid * EPT + win * EPW, 8)
            c1 = pltpu.async_copy(src_hbm.at[pl.ds(base, EPW)], srcv,
                                  gsem[0])
            c2 = pltpu.async_copy(w_hbm.at[pl.ds(base, EPW)], wv, gsem[1])
            c3 = pltpu.async_copy(dst_hbm.at[pl.ds(base, EPW)], dstv,
                                  gsem[2])
            c1.wait()
            c2.wait()
            c3.wait()

        stage(0)
        pltpu.async_copy(h_hbm.at[srcv.at[pl.ds(0, CH)]], rows[0], gsem[0])
        pltpu.async_copy(h_hbm.at[srcv.at[pl.ds(CH, CH)]], rows[1], gsem[1])
        rbase = pl.multiple_of(s * RPT, 8)

        @pl.when(c == 0)
        def _():
            @pl.when(s < NS - 1)
            def _():
                pltpu.sync_copy(out0_hbm.at[pl.ds(rbase, RPT)],
                                acc.at[pl.ds(rbase, RPT)])

            @pl.when(s == NS - 1)
            def _():
                pltpu.sync_copy(
                    out0_hbm.at[pl.ds((NS - 1) * RPT, RPT_LAST)],
                    acc.at[pl.ds((NS - 1) * RPT, RPT_LAST)])

        @pl.when(c == 1)
        def _():
            zv = jnp.zeros((L,), jnp.float32)
            for r in range(CH):
                for j in range(D // L):
                    rows[2][r, pl.ds(j * L, L)] = zv

            def zfill(base0, n):
                for i in range(n // CH):
                    pltpu.sync_copy(rows[2],
                                    acc.at[pl.ds(base0 + i * CH, CH)])
                t = n % CH
                if t:
                    pltpu.sync_copy(
                        rows[2].at[pl.ds(0, t)],
                        acc.at[pl.ds(base0 + (n // CH) * CH, t)])

            @pl.when(s < NS - 1)
            def _():
                zfill(rbase, RPT)

            @pl.when(s == NS - 1)
            def _():
                zfill((NS - 1) * RPT, RPT_LAST)

        plsc.subcore_barrier()

        def issue_gather(k, b):
            pltpu.async_copy(
                h_hbm.at[srcv.at[pl.ds(k * CH, CH)]], rows[b], gsem[b])

        def wait_gather(k, b):
            pltpu.make_async_copy(
                h_hbm.at[srcv.at[pl.ds(k * CH, CH)]], rows[b],
                gsem[b]).wait()

        def fill_dbuf(k, b):
            for q in range(CH // L):
                dbuf[b][pl.ds(q * L, L)] = dstv[pl.ds(k * CH + q * L, L)]

        def issue_scatter(k, b):
            pltpu.async_copy(rows[b], acc.at[dbuf[b]], csem[b], add=True)

        def wait_scatter(b):
            pltpu.make_async_copy(rows[b], acc.at[dbuf[b]], csem[b]).wait()

        def scale(k, b):
            def group_body(g, carry2):
                w16 = wv[pl.ds(k * CH + g * L, L)]
                for e in range(L):
                    wb = jnp.full((L,), w16[e], jnp.float32)
                    row = g * L + e
                    for j in range(D // L):
                        sl = pl.ds(j * L, L)
                        rows[b][row, sl] = rows[b][row, sl] * wb
                return carry2

            lax.fori_loop(0, CH // L, group_body, 0)

        def quad_body(q, carry):
            k0 = q * 4
            for b in range(4):
                k = k0 + b
                b2 = (b + 2) % 4
                wait_gather(k, b)

                @pl.when(k >= 2)
                def _():
                    wait_scatter(b2)

                @pl.when(k < WCH - 2)
                def _():
                    issue_gather(k + 2, b2)

                fill_dbuf(k, b)
                scale(k, b)
                issue_scatter(k, b)
            return carry

        for win in range(NW_):
            if win > 0:
                stage(win)
                issue_gather(0, 0)
                issue_gather(1, 1)
            lax.fori_loop(0, WCH // 4, quad_body, 0)
            wait_scatter((WCH - 2) % 4)
            wait_scatter((WCH - 1) % 4)
        plsc.subcore_barrier()

        @pl.when(s < NS - 1)
        def _():
            pltpu.sync_copy(acc.at[pl.ds(rbase, RPT)],
                            out_hbm.at[c].at[pl.ds(rbase, RPT)])

        @pl.when(s == NS - 1)
        def _():
            pltpu.sync_copy(
                acc.at[pl.ds((NS - 1) * RPT, RPT_LAST)],
                out_hbm.at[c].at[pl.ds((NS - 1) * RPT, RPT_LAST)])

    return sc_kernel


def kernel(x_src, x_dst, edge_index_sd, edge_weight, W_nei, W_self, b_self):
    N_SRC, D = x_src.shape
    N_DST = x_dst.shape[0]
    E = edge_weight.shape[0]
    NW = NC * NS

    EPT = ((E + NW * CH * 8 - 1) // (NW * CH * 8)) * CH * 8
    E_pad = EPT * NW
    src = edge_index_sd[0].astype(jnp.int32)
    dst = edge_index_sd[1].astype(jnp.int32)
    pad = E_pad - E
    pad_idx = jnp.arange(pad, dtype=jnp.int32)
    src_p = jnp.concatenate([src, pad_idx % N_SRC])
    dst_p = jnp.concatenate([dst, pad_idx % N_DST])
    w_p = jnp.concatenate([edge_weight, jnp.zeros((pad,), jnp.float32)])

    BN = 1000
    h, out0 = pl.pallas_call(
        _tc_matmul_body,
        grid=(N_SRC // BN,),
        in_specs=[
            pl.BlockSpec((BN, D), lambda i: (i, 0)),
            pl.BlockSpec((BN, D), lambda i: (i, 0)),
            pl.BlockSpec((D, D), lambda i: (0, 0)),
            pl.BlockSpec((D, D), lambda i: (0, 0)),
            pl.BlockSpec((1, D), lambda i: (0, 0)),
        ],
        out_specs=[
            pl.BlockSpec((BN, D), lambda i: (i, 0)),
            pl.BlockSpec((BN, D), lambda i: (i, 0)),
        ],
        out_shape=[
            jax.ShapeDtypeStruct((N_SRC, D), jnp.float32),
            jax.ShapeDtypeStruct((N_DST, D), jnp.float32),
        ],
    )(x_src, x_dst, W_nei, W_self, b_self.reshape(1, D))

    sc_kernel = _make_sc_kernel(N_DST, D, EPT)
    partials = sc_kernel(h, out0, src_p, w_p, dst_p)

    out = pl.pallas_call(
        _combine_body,
        grid=(N_DST // BN,),
        in_specs=[pl.BlockSpec((NC, BN, D), lambda i: (0, i, 0))],
        out_specs=pl.BlockSpec((BN, D), lambda i: (i, 0)),
        out_shape=jax.ShapeDtypeStruct((N_DST, D), jnp.float32),
    )(partials)
    return out
